# Initial kernel scaffold; baseline (speedup 1.0000x reference)
import jax
import jax.numpy as jnp
from jax.experimental import pallas as pl


def kernel(x, edge_index, W1, W2, final_weight):
    raise NotImplementedError("write your pallas kernel here")


# speedup vs baseline: 128.5920x; 128.5920x over previous
import functools

import jax
import jax.numpy as jnp
from jax import lax
from jax.experimental import pallas as pl
from jax.experimental.pallas import tpu as pltpu
from jax.experimental.pallas import tpu_sc as plsc

N = 100000
E = 6400000
ROWS = E // 128
N_PAIRS = 16
Q = 16
NCHUNKS = ROWS // Q
EPS = 1e-15


def _sc_scatter(table, src_rows, dst_rows, zeros):
    mesh = plsc.VectorSubcoreMesh(
        core_axis_name="c", subcore_axis_name="s", num_cores=2, num_subcores=16
    )

    @functools.partial(
        pl.kernel,
        out_type=jax.ShapeDtypeStruct((2, 2, N), jnp.float32),
        mesh=mesh,
        scratch_types=[
            pltpu.VMEM((N,), jnp.float32),
            pltpu.VMEM((2, Q, 128), jnp.int32),
            pltpu.VMEM((2, Q, 128), jnp.int32),
            pltpu.VMEM((2, Q, 128), jnp.float32),
            pltpu.VMEM_SHARED((N,), jnp.float32),
            pltpu.VMEM_SHARED((N,), jnp.float32),
            pltpu.SemaphoreType.DMA,
            pltpu.SemaphoreType.DMA,
        ],
        compiler_params=pltpu.CompilerParams(needs_layout_passes=False),
    )
    def sc_fn(t_hbm, src_hbm, dst_hbm, zero_hbm, out_hbm,
              tbl, srcb, dstb, vals, acc0, acc1, in_sem, sc_sem):
        c = lax.axis_index("c")
        s = lax.axis_index("s")
        f = s % 2
        gp = c * 8 + s // 2

        pltpu.sync_copy(t_hbm.at[f], tbl)

        @pl.when(s == 0)
        def _():
            pltpu.sync_copy(zero_hbm, acc0)

        @pl.when(s == 1)
        def _():
            pltpu.sync_copy(zero_hbm, acc1)

        plsc.subcore_barrier()

        n_chunks = jnp.where(gp < NCHUNKS % N_PAIRS,
                             NCHUNKS // N_PAIRS + 1, NCHUNKS // N_PAIRS)

        def row_of(i):
            return (gp + i * N_PAIRS) * Q

        def start_in(i, b):
            pltpu.async_copy(src_hbm.at[pl.ds(row_of(i), Q)], srcb.at[b], in_sem)
            pltpu.async_copy(dst_hbm.at[pl.ds(row_of(i), Q)], dstb.at[b], in_sem)

        def wait_in(b):
            pltpu.make_async_copy(src_hbm.at[pl.ds(0, Q)], srcb.at[b], in_sem).wait()
            pltpu.make_async_copy(dst_hbm.at[pl.ds(0, Q)], dstb.at[b], in_sem).wait()

        def drain_scatters(b):
            for j in range(Q):
                pltpu.make_async_copy(
                    vals.at[b].at[j], acc0.at[dstb.at[b].at[j]], sc_sem).wait()

        start_in(0, 0)

        def body(i, carry):
            b = lax.rem(i, 2)
            wait_in(b)

            for j in range(Q):
                for k in range(8):
                    idx = srcb[b, j, k * 16:(k + 1) * 16]
                    vals[b, j, k * 16:(k + 1) * 16] = plsc.load_gather(tbl, [idx])

            @pl.when(i > 0)
            def _():
                drain_scatters(1 - b)

            @pl.when(i < n_chunks - 1)
            def _():
                start_in(i + 1, 1 - b)

            @pl.when(f == 0)
            def _():
                for j in range(Q):
                    pltpu.async_copy(vals.at[b].at[j],
                                     acc0.at[dstb.at[b].at[j]], sc_sem, add=True)

            @pl.when(f == 1)
            def _():
                for j in range(Q):
                    pltpu.async_copy(vals.at[b].at[j],
                                     acc1.at[dstb.at[b].at[j]], sc_sem, add=True)

            return carry

        lax.fori_loop(0, n_chunks, body, 0)
        drain_scatters(lax.rem(n_chunks - 1, 2))

        plsc.subcore_barrier()

        @pl.when(s == 0)
        def _():
            pltpu.sync_copy(acc0, out_hbm.at[c, 0])

        @pl.when(s == 1)
        def _():
            pltpu.sync_copy(acc1, out_hbm.at[c, 1])

    return sc_fn(table, src_rows, dst_rows, zeros)


def _tc_prepare(xt, w):
    def body(x_ref, w_ref, o_ref):
        x = x_ref[...]
        nrm = jnp.sqrt(jnp.sum(x * x, axis=0, keepdims=True))
        xn = x / (nrm + EPS)
        x0 = xn[0:1, :]
        x1 = xn[1:2, :]
        o_ref[0:1, :] = x0 * w_ref[0, 0] + x1 * w_ref[1, 0]
        o_ref[1:2, :] = x0 * w_ref[0, 1] + x1 * w_ref[1, 1]

    return pl.pallas_call(
        body,
        out_shape=jax.ShapeDtypeStruct((2, N), jnp.float32),
        in_specs=[
            pl.BlockSpec(memory_space=pltpu.VMEM),
            pl.BlockSpec(memory_space=pltpu.SMEM),
        ],
        out_specs=pl.BlockSpec(memory_space=pltpu.VMEM),
    )(xt, w)


def _tc_mid(parts, w):
    def body(p_ref, w_ref, o_ref):
        p = p_ref[...]
        a = p[0] + p[1]
        x = jnp.maximum(a, 0.0)
        nrm = jnp.sqrt(jnp.sum(x * x, axis=0, keepdims=True))
        xn = x / (nrm + EPS)
        x0 = xn[0:1, :]
        x1 = xn[1:2, :]
        o_ref[0:1, :] = x0 * w_ref[0, 0] + x1 * w_ref[1, 0]
        o_ref[1:2, :] = x0 * w_ref[0, 1] + x1 * w_ref[1, 1]

    return pl.pallas_call(
        body,
        out_shape=jax.ShapeDtypeStruct((2, N), jnp.float32),
        in_specs=[
            pl.BlockSpec(memory_space=pltpu.VMEM),
            pl.BlockSpec(memory_space=pltpu.SMEM),
        ],
        out_specs=pl.BlockSpec(memory_space=pltpu.VMEM),
    )(parts, w)


def _tc_final(parts, fw):
    def body(p_ref, fw_ref, o_ref):
        p = p_ref[...]
        a = p[0] + p[1]
        x = jnp.maximum(a, 0.0)
        z = x[0:1, :] * fw_ref[0] + x[1:2, :] * fw_ref[1]
        o_ref[...] = 1.0 / (1.0 + jnp.exp(-z))

    return pl.pallas_call(
        body,
        out_shape=jax.ShapeDtypeStruct((1, N), jnp.float32),
        in_specs=[
            pl.BlockSpec(memory_space=pltpu.VMEM),
            pl.BlockSpec(memory_space=pltpu.SMEM),
        ],
        out_specs=pl.BlockSpec(memory_space=pltpu.VMEM),
    )(parts, fw)


def kernel(x, edge_index, W1, W2, final_weight):
    xt = x.T
    src_rows = edge_index[0].reshape(ROWS, 128)
    dst_rows = edge_index[1].reshape(ROWS, 128)
    zeros = jnp.zeros((N,), jnp.float32)

    t1 = _tc_prepare(xt, W1)
    p1 = _sc_scatter(t1, src_rows, dst_rows, zeros)
    t2 = _tc_mid(p1, W2)
    p2 = _sc_scatter(t2, src_rows, dst_rows, zeros)
    out = _tc_final(p2, final_weight)
    return out[0]
